# baseline (speedup 1.0000x reference)
import functools

import jax
import jax.numpy as jnp
from jax import lax
from jax.experimental import pallas as pl
from jax.experimental.pallas import tpu as pltpu
from jax.experimental.pallas import tpu_sc as plsc

N = 4096
D = 1024
E = 8
K = 2

BR = 512
BT = 256
P = N * K + E * BT
NB = P // BT

NC = 2
NS = 16
NW = NC * NS
PAIRS = N * K
PPW = PAIRS // NW
CH = 32
NCH = PPW // CH
NBUF = 3
TW = N // NW


def _router_body(x_ref, wn_ref, bnr_ref, idx_ref, gate_ref, rank_ref,
                 cnt_ref, acc_ref):
    i = pl.program_id(0)
    x = x_ref[...]
    nl = lax.dot_general(x, wn_ref[...], (((1,), (1,)), ((), ())),
                         preferred_element_type=jnp.float32) + bnr_ref[...]
    col = lax.broadcasted_iota(jnp.int32, nl.shape, 1)
    v1 = jnp.max(nl, axis=1, keepdims=True)
    i1 = jnp.min(jnp.where(nl == v1, col, E), axis=1, keepdims=True)
    nl2 = jnp.where(col == i1, -jnp.inf, nl)
    v2 = jnp.max(nl2, axis=1, keepdims=True)
    i2 = jnp.min(jnp.where(nl2 == v2, col, E), axis=1, keepdims=True)
    e2 = jnp.exp(v2 - v1)
    denom = 1.0 + e2
    idx_ref[...] = jnp.concatenate([i1, i2], axis=1)
    gate_ref[...] = jnp.concatenate([1.0 / denom, e2 / denom], axis=1)

    oh1 = (col == i1).astype(jnp.float32)
    oh2 = (col == i2).astype(jnp.float32)
    oh = oh1 + oh2

    @pl.when(i == 0)
    def _():
        acc_ref[...] = jnp.zeros_like(acc_ref)

    row = lax.broadcasted_iota(jnp.int32, (BR, BR), 0)
    colr = lax.broadcasted_iota(jnp.int32, (BR, BR), 1)
    tril = (row > colr).astype(jnp.float32)
    cum = lax.dot_general(tril, oh, (((1,), (0,)), ((), ())),
                          preferred_element_type=jnp.float32) + acc_ref[...]
    r1 = jnp.sum(cum * oh1, axis=1, keepdims=True)
    r2 = jnp.sum(cum * oh2, axis=1, keepdims=True)
    rank_ref[...] = jnp.concatenate([r1, r2], axis=1).astype(jnp.int32)
    acc_ref[...] += jnp.sum(oh, axis=0, keepdims=True)
    cnt_ref[...] = acc_ref[...].astype(jnp.int32)


def _router(x, Wn, bnr):
    return pl.pallas_call(
        _router_body,
        grid=(N // BR,),
        in_specs=[
            pl.BlockSpec((BR, D), lambda i: (i, 0)),
            pl.BlockSpec((E, D), lambda i: (0, 0)),
            pl.BlockSpec((1, E), lambda i: (0, 0)),
        ],
        out_specs=[
            pl.BlockSpec((BR, K), lambda i: (i, 0)),
            pl.BlockSpec((BR, K), lambda i: (i, 0)),
            pl.BlockSpec((BR, K), lambda i: (i, 0)),
            pl.BlockSpec((1, E), lambda i: (0, 0)),
        ],
        out_shape=[
            jax.ShapeDtypeStruct((N, K), jnp.int32),
            jax.ShapeDtypeStruct((N, K), jnp.float32),
            jax.ShapeDtypeStruct((N, K), jnp.int32),
            jax.ShapeDtypeStruct((1, E), jnp.int32),
        ],
        scratch_shapes=[pltpu.VMEM((1, E), jnp.float32)],
    )(x, Wn, bnr)


@functools.cache
def _get_sc_dispatch():
    mesh = plsc.VectorSubcoreMesh(
        core_axis_name="c", subcore_axis_name="s",
        num_cores=NC, num_subcores=NS)
    return functools.partial(
        pl.kernel,
        out_type=jax.ShapeDtypeStruct((P, D), jnp.float32),
        mesh=mesh,
        scratch_types=(
            [pltpu.VMEM((NCH, CH), jnp.int32),
             pltpu.VMEM((NCH, CH), jnp.int32)]
            + [pltpu.VMEM((CH, D), jnp.float32) for _ in range(NBUF)]
            + [pltpu.SemaphoreType.DMA for _ in range(2 * NBUF)]
        ),
    )(_sc_dispatch_body)


def _sc_dispatch_body(tok_hbm, pos_hbm, x_hbm, out_hbm, tok_v, pos_v, *rest):
    bufs = rest[:NBUF]
    gsems = rest[NBUF:2 * NBUF]
    osems = rest[2 * NBUF:3 * NBUF]
    wid = lax.axis_index("s") * NC + lax.axis_index("c")
    pltpu.sync_copy(tok_hbm.at[wid], tok_v)
    pltpu.sync_copy(pos_hbm.at[wid], pos_v)
    gcp = [None] * NBUF
    ocp = [None] * NBUF
    for g in range(min(NBUF - 1, NCH)):
        gcp[g % NBUF] = pltpu.async_copy(
            x_hbm.at[tok_v.at[g]], bufs[g % NBUF], gsems[g % NBUF])
    for c in range(NCH):
        g = c + NBUF - 1
        if g < NCH:
            b2 = g % NBUF
            if ocp[b2] is not None:
                ocp[b2].wait()
                ocp[b2] = None
            gcp[b2] = pltpu.async_copy(
                x_hbm.at[tok_v.at[g]], bufs[b2], gsems[b2])
        b = c % NBUF
        gcp[b].wait()
        ocp[b] = pltpu.async_copy(
            bufs[b], out_hbm.at[pos_v.at[c]], osems[b])
    for b in range(NBUF):
        if ocp[b] is not None:
            ocp[b].wait()


def _mm_body(me_ref, mx_ref, xg_ref, w1_ref, b1_ref, w2_ref, b2_ref, out_ref):
    i = pl.program_id(0)

    @pl.when(mx_ref[i] == i)
    def _():
        xg = xg_ref[...].astype(jnp.bfloat16)
        w1 = w1_ref[0].astype(jnp.bfloat16)
        h = jnp.maximum(
            lax.dot_general(xg, w1, (((1,), (1,)), ((), ())),
                            preferred_element_type=jnp.float32) + b1_ref[0],
            0.0)
        eo = jnp.sum(h * w2_ref[0], axis=1, keepdims=True) + b2_ref[0, 0, 0]
        out_ref[...] = eo


def _block_mm(me, mx, xg, W1, b1r, w2r, b2r):
    grid_spec = pltpu.PrefetchScalarGridSpec(
        num_scalar_prefetch=2,
        grid=(NB,),
        in_specs=[
            pl.BlockSpec((BT, D), lambda i, me, mx: (mx[i], 0)),
            pl.BlockSpec((1, D, D), lambda i, me, mx: (me[i], 0, 0)),
            pl.BlockSpec((1, 1, D), lambda i, me, mx: (me[i], 0, 0)),
            pl.BlockSpec((1, 1, D), lambda i, me, mx: (me[i], 0, 0)),
            pl.BlockSpec((1, 1, 1), lambda i, me, mx: (me[i], 0, 0)),
        ],
        out_specs=pl.BlockSpec((BT, 1), lambda i, me, mx: (i, 0)),
    )
    return pl.pallas_call(
        _mm_body,
        grid_spec=grid_spec,
        out_shape=jax.ShapeDtypeStruct((P, 1), jnp.float32),
    )(me, mx, xg, W1, b1r, w2r, b2r)


@functools.cache
def _get_sc_combine():
    mesh = plsc.VectorSubcoreMesh(
        core_axis_name="c", subcore_axis_name="s",
        num_cores=NC, num_subcores=NS)
    return functools.partial(
        pl.kernel,
        out_type=jax.ShapeDtypeStruct((N,), jnp.float32),
        mesh=mesh,
        compiler_params=pltpu.CompilerParams(needs_layout_passes=False),
        scratch_types=[
            pltpu.VMEM((P,), jnp.float32),
            pltpu.VMEM((TW,), jnp.int32),
            pltpu.VMEM((TW,), jnp.int32),
            pltpu.VMEM((TW,), jnp.float32),
            pltpu.VMEM((TW,), jnp.float32),
            pltpu.VMEM((TW,), jnp.float32),
        ],
    )(_sc_combine_body)


def _sc_combine_body(contrib_hbm, inv0_hbm, inv1_hbm, g0_hbm, g1_hbm, out_hbm,
                     c_v, i0_v, i1_v, g0_v, g1_v, o_v):
    wid = lax.axis_index("s") * NC + lax.axis_index("c")
    base = wid * TW
    pltpu.sync_copy(contrib_hbm, c_v)
    pltpu.sync_copy(inv0_hbm.at[pl.ds(base, TW)], i0_v)
    pltpu.sync_copy(inv1_hbm.at[pl.ds(base, TW)], i1_v)
    pltpu.sync_copy(g0_hbm.at[pl.ds(base, TW)], g0_v)
    pltpu.sync_copy(g1_hbm.at[pl.ds(base, TW)], g1_v)
    for j in range(TW // 16):
        s = pl.ds(j * 16, 16)
        a = plsc.load_gather(c_v, [i0_v[s]])
        b = plsc.load_gather(c_v, [i1_v[s]])
        o_v[s] = a * g0_v[s] + b * g1_v[s]
    pltpu.sync_copy(o_v, out_hbm.at[pl.ds(base, TW)])


def kernel(x, Wr, br, Wn, bn, W1, b1, W2, b2):
    del Wr, br
    bnr = bn.reshape(1, E)
    b1r = b1.reshape(E, 1, D)
    w2r = W2.reshape(E, 1, D)
    b2r = b2.reshape(E, 1, 1)

    idx, gate, rank, cnt = _router(x, Wn, bnr)

    e_flat = idx.reshape(-1)
    r_flat = rank.reshape(-1)
    counts = cnt.reshape(E)
    padded = ((counts + BT - 1) // BT) * BT
    pend = jnp.cumsum(padded)
    pstart = pend - padded
    pos = (pstart[e_flat] + r_flat).astype(jnp.int32)

    bi = jnp.arange(NB, dtype=jnp.int32)
    me_raw = jnp.minimum(
        jnp.searchsorted(pend, bi * BT, side="right"), E - 1).astype(jnp.int32)
    nb_used = pend[-1] // BT
    me = jnp.where(bi < nb_used, me_raw, me_raw[nb_used - 1])
    mx = jnp.where(bi < nb_used, bi, nb_used - 1).astype(jnp.int32)

    tok3 = jnp.repeat(jnp.arange(N, dtype=jnp.int32), K).reshape(NW, NCH, CH)
    xg = _get_sc_dispatch()(tok3, pos.reshape(NW, NCH, CH), x)
    contrib = _block_mm(me, mx, xg, W1, b1r, w2r, b2r)

    inv = pos.reshape(N, K)
    out = _get_sc_combine()(contrib.reshape(P), inv[:, 0], inv[:, 1],
                            gate[:, 0], gate[:, 1])
    return out.reshape(N, 1)


# speedup vs baseline: 1.0538x; 1.0538x over previous
import functools

import jax
import jax.numpy as jnp
from jax import lax
from jax.experimental import pallas as pl
from jax.experimental.pallas import tpu as pltpu
from jax.experimental.pallas import tpu_sc as plsc

N = 4096
D = 1024
E = 8
K = 2

BR = 512
BT = 256
BTS = 8
P = N * K + E * BT
NB = P // BT
NBP = 48

NC = 2
NS = 16
NW = NC * NS
PAIRS = N * K
PPW = PAIRS // NW
CH = 32
NCH = PPW // CH
NBUF = 2
TW = N // NW


def _router_body(x_ref, wn_ref, bnr_ref, code_ref, gate_ref,
                 cnt_ref, acc_ref):
    i = pl.program_id(0)
    x = x_ref[...]
    nl = lax.dot_general(x, wn_ref[...], (((1,), (1,)), ((), ())),
                         preferred_element_type=jnp.float32) + bnr_ref[...]
    col = lax.broadcasted_iota(jnp.int32, nl.shape, 1)
    v1 = jnp.max(nl, axis=1, keepdims=True)
    i1 = jnp.min(jnp.where(nl == v1, col, E), axis=1, keepdims=True)
    nl2 = jnp.where(col == i1, -jnp.inf, nl)
    v2 = jnp.max(nl2, axis=1, keepdims=True)
    i2 = jnp.min(jnp.where(nl2 == v2, col, E), axis=1, keepdims=True)
    e2 = jnp.exp(v2 - v1)
    denom = 1.0 + e2
    gate_ref[...] = jnp.concatenate([1.0 / denom, e2 / denom], axis=1)

    oh1 = (col == i1).astype(jnp.float32)
    oh2 = (col == i2).astype(jnp.float32)
    oh = oh1 + oh2

    @pl.when(i == 0)
    def _():
        acc_ref[...] = jnp.zeros_like(acc_ref)

    row = lax.broadcasted_iota(jnp.int32, (BR, BR), 0)
    colr = lax.broadcasted_iota(jnp.int32, (BR, BR), 1)
    tril = (row > colr).astype(jnp.float32)
    cum = lax.dot_general(tril, oh, (((1,), (0,)), ((), ())),
                          preferred_element_type=jnp.float32) + acc_ref[...]
    r1 = jnp.sum(cum * oh1, axis=1, keepdims=True).astype(jnp.int32)
    r2 = jnp.sum(cum * oh2, axis=1, keepdims=True).astype(jnp.int32)
    c1 = i1 + lax.shift_left(r1, 3)
    c2 = i2 + lax.shift_left(r2, 3)
    code_ref[...] = jnp.concatenate([c1, c2], axis=1)
    acc_ref[...] += jnp.sum(oh, axis=0, keepdims=True)
    cnt_ref[...] = jnp.concatenate(
        [acc_ref[...].astype(jnp.int32),
         jnp.zeros((1, 16 - E), jnp.int32)], axis=1)


def _router(x, Wn, bnr):
    return pl.pallas_call(
        _router_body,
        grid=(N // BR,),
        in_specs=[
            pl.BlockSpec((BR, D), lambda i: (i, 0)),
            pl.BlockSpec((E, D), lambda i: (0, 0)),
            pl.BlockSpec((1, E), lambda i: (0, 0)),
        ],
        out_specs=[
            pl.BlockSpec((BR, K), lambda i: (i, 0)),
            pl.BlockSpec((BR, K), lambda i: (i, 0)),
            pl.BlockSpec((1, 16), lambda i: (0, 0)),
        ],
        out_shape=[
            jax.ShapeDtypeStruct((N, K), jnp.int32),
            jax.ShapeDtypeStruct((N, K), jnp.float32),
            jax.ShapeDtypeStruct((1, 16), jnp.int32),
        ],
        scratch_shapes=[pltpu.VMEM((1, E), jnp.float32)],
    )(x, Wn, bnr)


@functools.cache
def _get_sc_plan():
    mesh = plsc.VectorSubcoreMesh(
        core_axis_name="c", subcore_axis_name="s",
        num_cores=NC, num_subcores=NS)
    return functools.partial(
        pl.kernel,
        out_type=[
            jax.ShapeDtypeStruct((NW, PPW), jnp.int32),
            jax.ShapeDtypeStruct((2, NBP), jnp.int32),
        ],
        mesh=mesh,
        compiler_params=pltpu.CompilerParams(needs_layout_passes=False),
        scratch_types=[
            pltpu.VMEM((PPW,), jnp.int32),
            pltpu.VMEM((16,), jnp.int32),
            pltpu.VMEM((16,), jnp.int32),
            pltpu.VMEM((16,), jnp.int32),
            pltpu.VMEM((PPW,), jnp.int32),
            pltpu.VMEM((2, NBP), jnp.int32),
        ],
    )(_sc_plan_body)


def _sc_plan_body(code_hbm, cnt_hbm, pos_hbm, meta_hbm,
                  code_v, cnt_v, ps_v, pe_v, pos_v, meta_v):
    wid = lax.axis_index("s") * NC + lax.axis_index("c")
    base_p = wid * PPW
    pltpu.sync_copy(code_hbm.at[pl.ds(base_p, PPW)], code_v)
    pltpu.sync_copy(cnt_hbm, cnt_v)

    iota = lax.iota(jnp.int32, 16)
    c16 = cnt_v[...]
    padded = jnp.bitwise_and(c16 + (BT - 1), -BT)
    padded = jnp.where(iota < E, padded, 0)
    incl = plsc.cumsum(padded)
    ps_v[...] = incl - padded
    pe_v[...] = incl

    for m in range(PPW // 16):
        s16 = pl.ds(m * 16, 16)
        code = code_v[s16]
        e = jnp.bitwise_and(code, E - 1)
        r = jnp.right_shift(code, 3)
        ps = plsc.load_gather(ps_v, [e])
        pos_v[s16] = ps + r
    pltpu.sync_copy(pos_v, pos_hbm.at[wid])

    @pl.when(wid == 0)
    def _():
        p7 = jnp.full((16,), E - 1, jnp.int32)
        nbv = jnp.right_shift(plsc.load_gather(pe_v, [p7]), BTS)
        s_last = lax.shift_left(nbv - 1, BTS)
        me_last = jnp.zeros((16,), jnp.int32)
        for e in range(E):
            pe = plsc.load_gather(pe_v, [jnp.full((16,), e, jnp.int32)])
            me_last += (pe <= s_last).astype(jnp.int32)
        me_last = jnp.minimum(me_last, E - 1)
        for g in range(NBP // 16):
            bi = g * 16 + iota
            s = lax.shift_left(bi, BTS)
            me = jnp.zeros((16,), jnp.int32)
            for e in range(E):
                pe = plsc.load_gather(pe_v, [jnp.full((16,), e, jnp.int32)])
                me += (pe <= s).astype(jnp.int32)
            me = jnp.minimum(me, E - 1)
            s16 = pl.ds(g * 16, 16)
            meta_v[0, s16] = jnp.where(bi < nbv, me, me_last)
            meta_v[1, s16] = jnp.where(bi < nbv, bi, nbv - 1)
        pltpu.sync_copy(meta_v, meta_hbm)


@functools.cache
def _get_sc_move():
    mesh = plsc.VectorSubcoreMesh(
        core_axis_name="c", subcore_axis_name="s",
        num_cores=NC, num_subcores=NS)
    return functools.partial(
        pl.kernel,
        out_type=jax.ShapeDtypeStruct((P, D), jnp.float32),
        mesh=mesh,
        scratch_types=(
            [pltpu.VMEM((NCH, CH), jnp.int32),
             pltpu.VMEM((NCH, CH), jnp.int32)]
            + [pltpu.VMEM((CH, D), jnp.float32) for _ in range(NBUF)]
            + [pltpu.SemaphoreType.DMA for _ in range(2 * NBUF)]
        ),
    )(_sc_move_body)


def _sc_move_body(tok_hbm, pos_hbm, x_hbm, out_hbm, tok_v, pos_v, *rest):
    bufs = rest[:NBUF]
    gsems = rest[NBUF:2 * NBUF]
    osems = rest[2 * NBUF:3 * NBUF]
    wid = lax.axis_index("s") * NC + lax.axis_index("c")
    pltpu.sync_copy(tok_hbm.at[wid], tok_v)
    pltpu.sync_copy(pos_hbm.at[wid], pos_v)
    gcp = [None] * NBUF
    ocp = [None] * NBUF
    for g in range(min(NBUF - 1, NCH)):
        gcp[g % NBUF] = pltpu.async_copy(
            x_hbm.at[tok_v.at[g]], bufs[g % NBUF], gsems[g % NBUF])
    for c in range(NCH):
        g = c + NBUF - 1
        if g < NCH:
            b2 = g % NBUF
            if ocp[b2] is not None:
                ocp[b2].wait()
                ocp[b2] = None
            gcp[b2] = pltpu.async_copy(
                x_hbm.at[tok_v.at[g]], bufs[b2], gsems[b2])
        b = c % NBUF
        gcp[b].wait()
        ocp[b] = pltpu.async_copy(
            bufs[b], out_hbm.at[pos_v.at[c]], osems[b])
    for b in range(NBUF):
        if ocp[b] is not None:
            ocp[b].wait()


def _mm_body(meta_ref, xg_ref, w1_ref, b1_ref, w2_ref, b2_ref, out_ref):
    i = pl.program_id(0)

    @pl.when(meta_ref[1, i] == i)
    def _():
        xg = xg_ref[...].astype(jnp.bfloat16)
        w1 = w1_ref[0].astype(jnp.bfloat16)
        h = jnp.maximum(
            lax.dot_general(xg, w1, (((1,), (1,)), ((), ())),
                            preferred_element_type=jnp.float32) + b1_ref[0],
            0.0)
        eo = jnp.sum(h * w2_ref[0], axis=1, keepdims=True) + b2_ref[0, 0, 0]
        out_ref[...] = eo


def _block_mm(meta, xg, W1, b1r, w2r, b2r):
    grid_spec = pltpu.PrefetchScalarGridSpec(
        num_scalar_prefetch=1,
        grid=(NB,),
        in_specs=[
            pl.BlockSpec((BT, D), lambda i, meta: (meta[1, i], 0)),
            pl.BlockSpec((1, D, D), lambda i, meta: (meta[0, i], 0, 0)),
            pl.BlockSpec((1, 1, D), lambda i, meta: (meta[0, i], 0, 0)),
            pl.BlockSpec((1, 1, D), lambda i, meta: (meta[0, i], 0, 0)),
            pl.BlockSpec((1, 1, 1), lambda i, meta: (meta[0, i], 0, 0)),
        ],
        out_specs=pl.BlockSpec((BT, 1), lambda i, meta: (i, 0)),
    )
    return pl.pallas_call(
        _mm_body,
        grid_spec=grid_spec,
        out_shape=jax.ShapeDtypeStruct((P, 1), jnp.float32),
    )(meta, xg, W1, b1r, w2r, b2r)


@functools.cache
def _get_sc_combine():
    mesh = plsc.VectorSubcoreMesh(
        core_axis_name="c", subcore_axis_name="s",
        num_cores=NC, num_subcores=NS)
    return functools.partial(
        pl.kernel,
        out_type=jax.ShapeDtypeStruct((N,), jnp.float32),
        mesh=mesh,
        compiler_params=pltpu.CompilerParams(needs_layout_passes=False),
        scratch_types=[
            pltpu.VMEM((P,), jnp.float32),
            pltpu.VMEM((PPW,), jnp.int32),
            pltpu.VMEM((PPW,), jnp.float32),
            pltpu.VMEM((TW,), jnp.float32),
        ],
    )(_sc_combine_body)


def _sc_combine_body(contrib_hbm, pos3_hbm, gate_hbm, out_hbm,
                     c_v, p_v, g_v, o_v):
    wid = lax.axis_index("s") * NC + lax.axis_index("c")
    base_t = wid * TW
    base_p = wid * PPW
    pltpu.sync_copy(contrib_hbm, c_v)
    pltpu.sync_copy(pos3_hbm.at[wid], p_v)
    pltpu.sync_copy(gate_hbm.at[pl.ds(base_p, PPW)], g_v)
    iota = lax.iota(jnp.int32, 16)
    for m in range(TW // 16):
        j0 = lax.shift_left(m * 16 + iota, 1)
        j1 = j0 + 1
        p0 = plsc.load_gather(p_v, [j0])
        p1 = plsc.load_gather(p_v, [j1])
        g0 = plsc.load_gather(g_v, [j0])
        g1 = plsc.load_gather(g_v, [j1])
        a = plsc.load_gather(c_v, [p0])
        b = plsc.load_gather(c_v, [p1])
        o_v[pl.ds(m * 16, 16)] = a * g0 + b * g1
    pltpu.sync_copy(o_v, out_hbm.at[pl.ds(base_t, TW)])


def kernel(x, Wr, br, Wn, bn, W1, b1, W2, b2):
    del Wr, br
    bnr = bn.reshape(1, E)
    b1r = b1.reshape(E, 1, D)
    w2r = W2.reshape(E, 1, D)
    b2r = b2.reshape(E, 1, 1)

    code, gate, cnt = _router(x, Wn, bnr)
    code1 = code.reshape(PAIRS)
    gate1 = gate.reshape(PAIRS)
    counts = cnt.reshape(16)[:E]
    padded = ((counts + BT - 1) // BT) * BT
    pend = jnp.cumsum(padded)
    pstart = pend - padded
    pos = (pstart[jnp.bitwise_and(code1, E - 1)]
           + jnp.right_shift(code1, 3)).astype(jnp.int32).reshape(NW, PPW)
    bi = jnp.arange(NBP, dtype=jnp.int32)
    me_raw = jnp.minimum(
        jnp.searchsorted(pend, bi * BT, side="right"), E - 1).astype(jnp.int32)
    nb_used = pend[-1] // BT
    me = jnp.where(bi < nb_used, me_raw, me_raw[nb_used - 1])
    mx = jnp.where(bi < nb_used, bi, nb_used - 1).astype(jnp.int32)
    meta = jnp.stack([me, mx])
    tok3 = jnp.repeat(jnp.arange(N, dtype=jnp.int32), K).reshape(NW, NCH, CH)
    xg = _get_sc_move()(tok3, pos.reshape(NW, NCH, CH), x)
    contrib = _block_mm(meta, xg, W1, b1r, w2r, b2r)
    return _get_sc_combine()(contrib.reshape(P), pos, gate1).reshape(N, 1)


# speedup vs baseline: 1.0994x; 1.0433x over previous
import functools

import jax
import jax.numpy as jnp
from jax import lax
from jax.experimental import pallas as pl
from jax.experimental.pallas import tpu as pltpu
from jax.experimental.pallas import tpu_sc as plsc

N = 4096
D = 1024
E = 8
K = 2

BR = 512
BT = 256
BTS = 8
P = N * K + E * BT
NB = P // BT
NBP = 48

NC = 2
NS = 16
NW = NC * NS
PAIRS = N * K
PPW = PAIRS // NW
CH = 32
NCH = PPW // CH
NBUF = 2
TW = N // NW


def _router_body(x_ref, wn_ref, bnr_ref, code_ref, gate_ref,
                 cnt_ref, acc_ref):
    i = pl.program_id(0)
    x = x_ref[...]
    nl = lax.dot_general(x, wn_ref[...], (((1,), (1,)), ((), ())),
                         preferred_element_type=jnp.float32) + bnr_ref[...]
    col = lax.broadcasted_iota(jnp.int32, nl.shape, 1)
    v1 = jnp.max(nl, axis=1, keepdims=True)
    i1 = jnp.min(jnp.where(nl == v1, col, E), axis=1, keepdims=True)
    nl2 = jnp.where(col == i1, -jnp.inf, nl)
    v2 = jnp.max(nl2, axis=1, keepdims=True)
    i2 = jnp.min(jnp.where(nl2 == v2, col, E), axis=1, keepdims=True)
    e2 = jnp.exp(v2 - v1)
    denom = 1.0 + e2
    gate_ref[...] = jnp.concatenate([1.0 / denom, e2 / denom], axis=1)

    oh1 = (col == i1).astype(jnp.float32)
    oh2 = (col == i2).astype(jnp.float32)
    oh = oh1 + oh2

    @pl.when(i == 0)
    def _():
        acc_ref[...] = jnp.zeros_like(acc_ref)

    row = lax.broadcasted_iota(jnp.int32, (BR, BR), 0)
    colr = lax.broadcasted_iota(jnp.int32, (BR, BR), 1)
    tril = (row > colr).astype(jnp.float32)
    cum = lax.dot_general(tril, oh, (((1,), (0,)), ((), ())),
                          preferred_element_type=jnp.float32) + acc_ref[...]
    r1 = jnp.sum(cum * oh1, axis=1, keepdims=True).astype(jnp.int32)
    r2 = jnp.sum(cum * oh2, axis=1, keepdims=True).astype(jnp.int32)
    c1 = i1 + lax.shift_left(r1, 3)
    c2 = i2 + lax.shift_left(r2, 3)
    code_ref[...] = jnp.concatenate([c1, c2], axis=1)
    acc_ref[...] += jnp.sum(oh, axis=0, keepdims=True)
    cnt_ref[...] = jnp.concatenate(
        [acc_ref[...].astype(jnp.int32),
         jnp.zeros((1, 16 - E), jnp.int32)], axis=1)


def _router(x, Wn, bnr):
    return pl.pallas_call(
        _router_body,
        grid=(N // BR,),
        in_specs=[
            pl.BlockSpec((BR, D), lambda i: (i, 0)),
            pl.BlockSpec((E, D), lambda i: (0, 0)),
            pl.BlockSpec((1, E), lambda i: (0, 0)),
        ],
        out_specs=[
            pl.BlockSpec((BR, K), lambda i: (i, 0)),
            pl.BlockSpec((BR, K), lambda i: (i, 0)),
            pl.BlockSpec((1, 16), lambda i: (0, 0)),
        ],
        out_shape=[
            jax.ShapeDtypeStruct((N, K), jnp.int32),
            jax.ShapeDtypeStruct((N, K), jnp.float32),
            jax.ShapeDtypeStruct((1, 16), jnp.int32),
        ],
        scratch_shapes=[pltpu.VMEM((1, E), jnp.float32)],
    )(x, Wn, bnr)


@functools.cache
def _get_sc_plan():
    mesh = plsc.VectorSubcoreMesh(
        core_axis_name="c", subcore_axis_name="s",
        num_cores=NC, num_subcores=NS)
    return functools.partial(
        pl.kernel,
        out_type=jax.ShapeDtypeStruct((NW, PPW), jnp.int32),
        mesh=mesh,
        compiler_params=pltpu.CompilerParams(needs_layout_passes=False),
        scratch_types=[
            pltpu.VMEM((PPW,), jnp.int32),
            pltpu.VMEM((16,), jnp.int32),
            pltpu.VMEM((16,), jnp.int32),
            pltpu.VMEM((PPW,), jnp.int32),
        ],
    )(_sc_plan_body)


def _sc_plan_body(code_hbm, cnt_hbm, pos_hbm, code_v, cnt_v, ps_v, pos_v):
    wid = lax.axis_index("s") * NC + lax.axis_index("c")
    base_p = wid * PPW
    pltpu.sync_copy(code_hbm.at[pl.ds(base_p, PPW)], code_v)
    pltpu.sync_copy(cnt_hbm, cnt_v)

    iota = lax.iota(jnp.int32, 16)
    c16 = cnt_v[...]
    padded = jnp.bitwise_and(c16 + (BT - 1), -BT)
    padded = jnp.where(iota < E, padded, 0)
    incl = plsc.cumsum(padded)
    ps_v[...] = incl - padded

    for m in range(PPW // 16):
        s16 = pl.ds(m * 16, 16)
        code = code_v[s16]
        e = jnp.bitwise_and(code, E - 1)
        r = jnp.right_shift(code, 3)
        ps = plsc.load_gather(ps_v, [e])
        pos_v[s16] = ps + r
    pltpu.sync_copy(pos_v, pos_hbm.at[wid])


@functools.cache
def _get_sc_move():
    mesh = plsc.VectorSubcoreMesh(
        core_axis_name="c", subcore_axis_name="s",
        num_cores=NC, num_subcores=NS)
    return functools.partial(
        pl.kernel,
        out_type=jax.ShapeDtypeStruct((P, D), jnp.float32),
        mesh=mesh,
        scratch_types=(
            [pltpu.VMEM((NCH, CH), jnp.int32),
             pltpu.VMEM((NCH, CH), jnp.int32)]
            + [pltpu.VMEM((CH, D), jnp.float32) for _ in range(NBUF)]
            + [pltpu.SemaphoreType.DMA for _ in range(2 * NBUF)]
        ),
    )(_sc_move_body)


def _sc_move_body(tok_hbm, pos_hbm, x_hbm, out_hbm, tok_v, pos_v, *rest):
    bufs = rest[:NBUF]
    gsems = rest[NBUF:2 * NBUF]
    osems = rest[2 * NBUF:3 * NBUF]
    wid = lax.axis_index("s") * NC + lax.axis_index("c")
    pltpu.sync_copy(tok_hbm.at[wid], tok_v)
    pltpu.sync_copy(pos_hbm.at[wid], pos_v)
    gcp = [None] * NBUF
    ocp = [None] * NBUF
    for g in range(min(NBUF - 1, NCH)):
        gcp[g % NBUF] = pltpu.async_copy(
            x_hbm.at[tok_v.at[g]], bufs[g % NBUF], gsems[g % NBUF])
    for c in range(NCH):
        g = c + NBUF - 1
        if g < NCH:
            b2 = g % NBUF
            if ocp[b2] is not None:
                ocp[b2].wait()
                ocp[b2] = None
            gcp[b2] = pltpu.async_copy(
                x_hbm.at[tok_v.at[g]], bufs[b2], gsems[b2])
        b = c % NBUF
        gcp[b].wait()
        ocp[b] = pltpu.async_copy(
            bufs[b], out_hbm.at[pos_v.at[c]], osems[b])
    for b in range(NBUF):
        if ocp[b] is not None:
            ocp[b].wait()


def _mm_body(me_ref, mx_ref, xg_ref, w1_ref, b1_ref, w2_ref, b2_ref, out_ref):
    i = pl.program_id(0)

    @pl.when(mx_ref[i] == i)
    def _():
        xg = xg_ref[...].astype(jnp.bfloat16)
        w1 = w1_ref[0].astype(jnp.bfloat16)
        h = jnp.maximum(
            lax.dot_general(xg, w1, (((1,), (1,)), ((), ())),
                            preferred_element_type=jnp.float32) + b1_ref[0],
            0.0)
        eo = jnp.sum(h * w2_ref[0], axis=1, keepdims=True) + b2_ref[0, 0, 0]
        out_ref[...] = eo


def _block_mm(me, mx, xg, W1, b1r, w2r, b2r):
    grid_spec = pltpu.PrefetchScalarGridSpec(
        num_scalar_prefetch=2,
        grid=(NB,),
        in_specs=[
            pl.BlockSpec((BT, D), lambda i, me, mx: (mx[i], 0)),
            pl.BlockSpec((1, D, D), lambda i, me, mx: (me[i], 0, 0)),
            pl.BlockSpec((1, 1, D), lambda i, me, mx: (me[i], 0, 0)),
            pl.BlockSpec((1, 1, D), lambda i, me, mx: (me[i], 0, 0)),
            pl.BlockSpec((1, 1, 1), lambda i, me, mx: (me[i], 0, 0)),
        ],
        out_specs=pl.BlockSpec((BT, 1), lambda i, me, mx: (i, 0)),
    )
    return pl.pallas_call(
        _mm_body,
        grid_spec=grid_spec,
        out_shape=jax.ShapeDtypeStruct((P, 1), jnp.float32),
    )(me, mx, xg, W1, b1r, w2r, b2r)


@functools.cache
def _get_sc_combine():
    mesh = plsc.VectorSubcoreMesh(
        core_axis_name="c", subcore_axis_name="s",
        num_cores=NC, num_subcores=NS)
    return functools.partial(
        pl.kernel,
        out_type=jax.ShapeDtypeStruct((N,), jnp.float32),
        mesh=mesh,
        compiler_params=pltpu.CompilerParams(needs_layout_passes=False),
        scratch_types=[
            pltpu.VMEM((P,), jnp.float32),
            pltpu.VMEM((PPW,), jnp.int32),
            pltpu.VMEM((PPW,), jnp.float32),
            pltpu.VMEM((TW,), jnp.float32),
        ],
    )(_sc_combine_body)


def _sc_combine_body(contrib_hbm, pos3_hbm, gate_hbm, out_hbm,
                     c_v, p_v, g_v, o_v):
    wid = lax.axis_index("s") * NC + lax.axis_index("c")
    base_t = wid * TW
    base_p = wid * PPW
    pltpu.sync_copy(contrib_hbm, c_v)
    pltpu.sync_copy(pos3_hbm.at[wid], p_v)
    pltpu.sync_copy(gate_hbm.at[pl.ds(base_p, PPW)], g_v)
    iota = lax.iota(jnp.int32, 16)
    for m in range(TW // 16):
        j0 = lax.shift_left(m * 16 + iota, 1)
        j1 = j0 + 1
        p0 = plsc.load_gather(p_v, [j0])
        p1 = plsc.load_gather(p_v, [j1])
        g0 = plsc.load_gather(g_v, [j0])
        g1 = plsc.load_gather(g_v, [j1])
        a = plsc.load_gather(c_v, [p0])
        b = plsc.load_gather(c_v, [p1])
        o_v[pl.ds(m * 16, 16)] = a * g0 + b * g1
    pltpu.sync_copy(o_v, out_hbm.at[pl.ds(base_t, TW)])


def kernel(x, Wr, br, Wn, bn, W1, b1, W2, b2):
    del Wr, br
    bnr = bn.reshape(1, E)
    b1r = b1.reshape(E, 1, D)
    w2r = W2.reshape(E, 1, D)
    b2r = b2.reshape(E, 1, 1)

    code, gate, cnt = _router(x, Wn, bnr)
    code1 = code.reshape(PAIRS)
    gate1 = gate.reshape(PAIRS)
    pos = _get_sc_plan()(code1, cnt.reshape(16))
    counts = cnt.reshape(16)[:E]
    padded = ((counts + BT - 1) // BT) * BT
    pend = jnp.cumsum(padded)
    bi = jnp.arange(NBP, dtype=jnp.int32)
    me_raw = jnp.minimum(
        jnp.searchsorted(pend, bi * BT, side="right"), E - 1).astype(jnp.int32)
    nb_used = pend[-1] // BT
    me = jnp.where(bi < nb_used, me_raw, me_raw[nb_used - 1])
    mx = jnp.where(bi < nb_used, bi, nb_used - 1).astype(jnp.int32)
    tok3 = jnp.repeat(jnp.arange(N, dtype=jnp.int32), K).reshape(NW, NCH, CH)
    xg = _get_sc_move()(tok3, pos.reshape(NW, NCH, CH), x)
    contrib = _block_mm(me, mx, xg, W1, b1r, w2r, b2r)
    return _get_sc_combine()(contrib.reshape(P), pos, gate1).reshape(N, 1)


# speedup vs baseline: 1.1089x; 1.0086x over previous
import functools

import jax
import jax.numpy as jnp
from jax import lax
from jax.experimental import pallas as pl
from jax.experimental.pallas import tpu as pltpu
from jax.experimental.pallas import tpu_sc as plsc

N = 4096
D = 1024
E = 8
K = 2

BR = 512
BT = 256
BTS = 8
P = N * K + E * BT
NB = P // BT
NBP = 48

NC = 2
NS = 16
NW = NC * NS
PAIRS = N * K
PPW = PAIRS // NW
CH = 32
NCH = PPW // CH
NBUF = 3
TW = N // NW


def _router_body(x_ref, wn_ref, bnr_ref, code_ref, gate_ref,
                 cnt_ref, acc_ref):
    i = pl.program_id(0)
    x = x_ref[...]
    nl = lax.dot_general(x, wn_ref[...], (((1,), (1,)), ((), ())),
                         preferred_element_type=jnp.float32) + bnr_ref[...]
    col = lax.broadcasted_iota(jnp.int32, nl.shape, 1)
    v1 = jnp.max(nl, axis=1, keepdims=True)
    i1 = jnp.min(jnp.where(nl == v1, col, E), axis=1, keepdims=True)
    nl2 = jnp.where(col == i1, -jnp.inf, nl)
    v2 = jnp.max(nl2, axis=1, keepdims=True)
    i2 = jnp.min(jnp.where(nl2 == v2, col, E), axis=1, keepdims=True)
    e2 = jnp.exp(v2 - v1)
    denom = 1.0 + e2
    gate_ref[...] = jnp.concatenate([1.0 / denom, e2 / denom], axis=1)

    oh1 = (col == i1).astype(jnp.float32)
    oh2 = (col == i2).astype(jnp.float32)
    oh = oh1 + oh2

    @pl.when(i == 0)
    def _():
        acc_ref[...] = jnp.zeros_like(acc_ref)

    row = lax.broadcasted_iota(jnp.int32, (BR, BR), 0)
    colr = lax.broadcasted_iota(jnp.int32, (BR, BR), 1)
    tril = (row > colr).astype(jnp.float32)
    cum = lax.dot_general(tril, oh, (((1,), (0,)), ((), ())),
                          preferred_element_type=jnp.float32) + acc_ref[...]
    r1 = jnp.sum(cum * oh1, axis=1, keepdims=True).astype(jnp.int32)
    r2 = jnp.sum(cum * oh2, axis=1, keepdims=True).astype(jnp.int32)
    c1 = i1 + lax.shift_left(r1, 3)
    c2 = i2 + lax.shift_left(r2, 3)
    code_ref[...] = jnp.concatenate([c1, c2], axis=1)
    acc_ref[...] += jnp.sum(oh, axis=0, keepdims=True)
    cnt_ref[...] = jnp.concatenate(
        [acc_ref[...].astype(jnp.int32),
         jnp.zeros((1, 16 - E), jnp.int32)], axis=1)


def _router(x, Wn, bnr):
    return pl.pallas_call(
        _router_body,
        grid=(N // BR,),
        in_specs=[
            pl.BlockSpec((BR, D), lambda i: (i, 0)),
            pl.BlockSpec((E, D), lambda i: (0, 0)),
            pl.BlockSpec((1, E), lambda i: (0, 0)),
        ],
        out_specs=[
            pl.BlockSpec((BR, K), lambda i: (i, 0)),
            pl.BlockSpec((BR, K), lambda i: (i, 0)),
            pl.BlockSpec((1, 16), lambda i: (0, 0)),
        ],
        out_shape=[
            jax.ShapeDtypeStruct((N, K), jnp.int32),
            jax.ShapeDtypeStruct((N, K), jnp.float32),
            jax.ShapeDtypeStruct((1, 16), jnp.int32),
        ],
        scratch_shapes=[pltpu.VMEM((1, E), jnp.float32)],
    )(x, Wn, bnr)


@functools.cache
def _get_sc_plan():
    mesh = plsc.VectorSubcoreMesh(
        core_axis_name="c", subcore_axis_name="s",
        num_cores=NC, num_subcores=NS)
    return functools.partial(
        pl.kernel,
        out_type=jax.ShapeDtypeStruct((NW, PPW), jnp.int32),
        mesh=mesh,
        compiler_params=pltpu.CompilerParams(needs_layout_passes=False),
        scratch_types=[
            pltpu.VMEM((PPW,), jnp.int32),
            pltpu.VMEM((16,), jnp.int32),
            pltpu.VMEM((16,), jnp.int32),
            pltpu.VMEM((PPW,), jnp.int32),
        ],
    )(_sc_plan_body)


def _sc_plan_body(code_hbm, cnt_hbm, pos_hbm, code_v, cnt_v, ps_v, pos_v):
    wid = lax.axis_index("s") * NC + lax.axis_index("c")
    base_p = wid * PPW
    pltpu.sync_copy(code_hbm.at[pl.ds(base_p, PPW)], code_v)
    pltpu.sync_copy(cnt_hbm, cnt_v)

    iota = lax.iota(jnp.int32, 16)
    c16 = cnt_v[...]
    padded = jnp.bitwise_and(c16 + (BT - 1), -BT)
    padded = jnp.where(iota < E, padded, 0)
    incl = plsc.cumsum(padded)
    ps_v[...] = incl - padded

    for m in range(PPW // 16):
        s16 = pl.ds(m * 16, 16)
        code = code_v[s16]
        e = jnp.bitwise_and(code, E - 1)
        r = jnp.right_shift(code, 3)
        ps = plsc.load_gather(ps_v, [e])
        pos_v[s16] = ps + r
    pltpu.sync_copy(pos_v, pos_hbm.at[wid])


@functools.cache
def _get_sc_move():
    mesh = plsc.VectorSubcoreMesh(
        core_axis_name="c", subcore_axis_name="s",
        num_cores=NC, num_subcores=NS)
    return functools.partial(
        pl.kernel,
        out_type=jax.ShapeDtypeStruct((P, D), jnp.float32),
        mesh=mesh,
        scratch_types=(
            [pltpu.VMEM((NCH, CH), jnp.int32),
             pltpu.VMEM((NCH, CH), jnp.int32)]
            + [pltpu.VMEM((CH, D), jnp.float32) for _ in range(NBUF)]
            + [pltpu.SemaphoreType.DMA for _ in range(2 * NBUF)]
        ),
    )(_sc_move_body)


def _sc_move_body(tok_hbm, pos_hbm, x_hbm, out_hbm, tok_v, pos_v, *rest):
    bufs = rest[:NBUF]
    gsems = rest[NBUF:2 * NBUF]
    osems = rest[2 * NBUF:3 * NBUF]
    wid = lax.axis_index("s") * NC + lax.axis_index("c")
    pltpu.sync_copy(tok_hbm.at[wid], tok_v)
    pltpu.sync_copy(pos_hbm.at[wid], pos_v)
    gcp = [None] * NBUF
    ocp = [None] * NBUF
    for g in range(min(NBUF - 1, NCH)):
        gcp[g % NBUF] = pltpu.async_copy(
            x_hbm.at[tok_v.at[g]], bufs[g % NBUF], gsems[g % NBUF])
    for c in range(NCH):
        g = c + NBUF - 1
        if g < NCH:
            b2 = g % NBUF
            if ocp[b2] is not None:
                ocp[b2].wait()
                ocp[b2] = None
            gcp[b2] = pltpu.async_copy(
                x_hbm.at[tok_v.at[g]], bufs[b2], gsems[b2])
        b = c % NBUF
        gcp[b].wait()
        ocp[b] = pltpu.async_copy(
            bufs[b], out_hbm.at[pos_v.at[c]], osems[b])
    for b in range(NBUF):
        if ocp[b] is not None:
            ocp[b].wait()


def _mm_body(me_ref, mx_ref, xg_ref, w1_ref, b1_ref, w2_ref, b2_ref, out_ref):
    i = pl.program_id(0)

    @pl.when(mx_ref[i] == i)
    def _():
        xg = xg_ref[...].astype(jnp.bfloat16)
        w1 = w1_ref[0].astype(jnp.bfloat16)
        h = jnp.maximum(
            lax.dot_general(xg, w1, (((1,), (1,)), ((), ())),
                            preferred_element_type=jnp.float32) + b1_ref[0],
            0.0)
        eo = jnp.sum(h * w2_ref[0], axis=1, keepdims=True) + b2_ref[0, 0, 0]
        out_ref[...] = eo


def _block_mm(me, mx, xg, W1, b1r, w2r, b2r):
    grid_spec = pltpu.PrefetchScalarGridSpec(
        num_scalar_prefetch=2,
        grid=(NB,),
        in_specs=[
            pl.BlockSpec((BT, D), lambda i, me, mx: (mx[i], 0)),
            pl.BlockSpec((1, D, D), lambda i, me, mx: (me[i], 0, 0)),
            pl.BlockSpec((1, 1, D), lambda i, me, mx: (me[i], 0, 0)),
            pl.BlockSpec((1, 1, D), lambda i, me, mx: (me[i], 0, 0)),
            pl.BlockSpec((1, 1, 1), lambda i, me, mx: (me[i], 0, 0)),
        ],
        out_specs=pl.BlockSpec((BT, 1), lambda i, me, mx: (i, 0)),
    )
    return pl.pallas_call(
        _mm_body,
        grid_spec=grid_spec,
        out_shape=jax.ShapeDtypeStruct((P, 1), jnp.float32),
    )(me, mx, xg, W1, b1r, w2r, b2r)


@functools.cache
def _get_sc_combine():
    mesh = plsc.VectorSubcoreMesh(
        core_axis_name="c", subcore_axis_name="s",
        num_cores=NC, num_subcores=NS)
    return functools.partial(
        pl.kernel,
        out_type=jax.ShapeDtypeStruct((N,), jnp.float32),
        mesh=mesh,
        compiler_params=pltpu.CompilerParams(needs_layout_passes=False),
        scratch_types=[
            pltpu.VMEM((P,), jnp.float32),
            pltpu.VMEM((PPW,), jnp.int32),
            pltpu.VMEM((PPW,), jnp.float32),
            pltpu.VMEM((TW,), jnp.float32),
        ],
    )(_sc_combine_body)


def _sc_combine_body(contrib_hbm, pos3_hbm, gate_hbm, out_hbm,
                     c_v, p_v, g_v, o_v):
    wid = lax.axis_index("s") * NC + lax.axis_index("c")
    base_t = wid * TW
    base_p = wid * PPW
    pltpu.sync_copy(contrib_hbm, c_v)
    pltpu.sync_copy(pos3_hbm.at[wid], p_v)
    pltpu.sync_copy(gate_hbm.at[pl.ds(base_p, PPW)], g_v)
    iota = lax.iota(jnp.int32, 16)
    for m in range(TW // 16):
        j0 = lax.shift_left(m * 16 + iota, 1)
        j1 = j0 + 1
        p0 = plsc.load_gather(p_v, [j0])
        p1 = plsc.load_gather(p_v, [j1])
        g0 = plsc.load_gather(g_v, [j0])
        g1 = plsc.load_gather(g_v, [j1])
        a = plsc.load_gather(c_v, [p0])
        b = plsc.load_gather(c_v, [p1])
        o_v[pl.ds(m * 16, 16)] = a * g0 + b * g1
    pltpu.sync_copy(o_v, out_hbm.at[pl.ds(base_t, TW)])


def kernel(x, Wr, br, Wn, bn, W1, b1, W2, b2):
    del Wr, br
    bnr = bn.reshape(1, E)
    b1r = b1.reshape(E, 1, D)
    w2r = W2.reshape(E, 1, D)
    b2r = b2.reshape(E, 1, 1)

    code, gate, cnt = _router(x, Wn, bnr)
    code1 = code.reshape(PAIRS)
    gate1 = gate.reshape(PAIRS)
    pos = _get_sc_plan()(code1, cnt.reshape(16))
    counts = cnt.reshape(16)[:E]
    padded = ((counts + BT - 1) // BT) * BT
    pend = jnp.cumsum(padded)
    bi = jnp.arange(NBP, dtype=jnp.int32)
    me_raw = jnp.minimum(
        jnp.searchsorted(pend, bi * BT, side="right"), E - 1).astype(jnp.int32)
    nb_used = pend[-1] // BT
    me = jnp.where(bi < nb_used, me_raw, me_raw[nb_used - 1])
    mx = jnp.where(bi < nb_used, bi, nb_used - 1).astype(jnp.int32)
    tok3 = jnp.repeat(jnp.arange(N, dtype=jnp.int32), K).reshape(NW, NCH, CH)
    xg = _get_sc_move()(tok3, pos.reshape(NW, NCH, CH), x)
    contrib = _block_mm(me, mx, xg, W1, b1r, w2r, b2r)
    return _get_sc_combine()(contrib.reshape(P), pos, gate1).reshape(N, 1)
